# Initial kernel scaffold; baseline (speedup 1.0000x reference)
import jax
import jax.numpy as jnp
from jax.experimental import pallas as pl


def kernel(x_age, x_sex, edge_index_has_age, edge_index_has_sex, W1_age, b1_age, W1_sex, b1_sex, W2_age, b2_age, W2_sex, b2_sex):
    raise NotImplementedError("write your pallas kernel here")


# speedup vs baseline: 3.7560x; 3.7560x over previous
import functools

import jax
import jax.numpy as jnp
from jax import lax
from jax.experimental import pallas as pl
from jax.experimental.pallas import tpu as pltpu
from jax.experimental.pallas import tpu_sc as plsc

N = 10000
E = 320000
F = 128
NC = 2
NS = 16
L = 16

CH = 128
GSZ = 16
GROUPS = 10
CHUNKS = GSZ * GROUPS
EPT = CH * CHUNKS
EPAD = NS * EPT
ACC_ROWS = 10240
ZROWS = ACC_ROWS // NS
OFF = ACC_ROWS

DPT = E // NS
DVECS = DPT // L
DROWS = 128

_mesh = plsc.VectorSubcoreMesh(core_axis_name="c", subcore_axis_name="s",
                               num_cores=NC, num_subcores=NS)


def _z16():
    return jnp.zeros((L,), jnp.float32)


@functools.partial(
    pl.kernel,
    out_type=jax.ShapeDtypeStruct((4 * DROWS, DROWS), jnp.float32),
    mesh=_mesh,
    scratch_types=[
        pltpu.VMEM((DPT,), jnp.int32),
        pltpu.VMEM((DPT,), jnp.int32),
        pltpu.VMEM((DROWS, DROWS), jnp.float32),
        pltpu.VMEM((DROWS, DROWS), jnp.float32),
        pltpu.VMEM((1, DROWS), jnp.int32),
        pltpu.VMEM_SHARED((DROWS, DROWS), jnp.float32),
        pltpu.VMEM_SHARED((DROWS, DROWS), jnp.float32),
    ],
    compiler_params=pltpu.CompilerParams(needs_layout_passes=False),
)
def _deg_kernel(dsrc_h, ddst_h, out_h, srcv, dstv, accs, accd, idv, sh_s, sh_d):
    c = lax.axis_index("c")
    s = lax.axis_index("s")
    w = c * NS + s

    z16 = _z16()

    def _zero(i, _):
        accs[i // 8, pl.ds((i % 8) * L, L)] = z16
        accd[i // 8, pl.ds((i % 8) * L, L)] = z16
        return 0

    lax.fori_loop(0, DROWS * 8, _zero, 0)

    iota = lax.iota(jnp.int32, L)
    for k in range(8):
        idv[0, pl.ds(k * L, L)] = k * L + iota

    zr = DROWS // NS
    pltpu.sync_copy(accs.at[pl.ds(0, zr)], sh_s.at[pl.ds(s * zr, zr)])
    pltpu.sync_copy(accd.at[pl.ds(0, zr)], sh_d.at[pl.ds(s * zr, zr)])

    pltpu.sync_copy(dsrc_h.at[w], srcv)
    pltpu.sync_copy(ddst_h.at[w], dstv)
    plsc.subcore_barrier()

    ones = jnp.ones((L,), jnp.float32)

    def _scat(acc, idx):
        row = lax.shift_right_logical(idx, 7)
        col = lax.bitwise_and(idx, 127)
        plsc.addupdate_scatter(acc, [row, col], ones)

    def _hist(i, _):
        _scat(accs, srcv[pl.ds(i * L, L)])
        _scat(accd, dstv[pl.ds(i * L, L)])
        return 0

    lax.fori_loop(0, DVECS, _hist, 0)

    pltpu.sync_copy(accs, sh_s.at[idv.at[0]], add=True)
    pltpu.sync_copy(accd, sh_d.at[idv.at[0]], add=True)
    plsc.subcore_barrier()

    base = c * (2 * DROWS)
    pltpu.sync_copy(sh_s.at[pl.ds(s * zr, zr)], out_h.at[pl.ds(base + s * zr, zr)])
    pltpu.sync_copy(sh_d.at[pl.ds(s * zr, zr)],
                    out_h.at[pl.ds(base + DROWS + s * zr, zr)])


@functools.partial(
    pl.kernel,
    out_type=jax.ShapeDtypeStruct((2 * ACC_ROWS, F), jnp.float32),
    mesh=_mesh,
    scratch_types=[
        pltpu.VMEM((GSZ, CH), jnp.int32),
        pltpu.VMEM((GSZ, CH), jnp.int32),
        pltpu.VMEM((CH, F), jnp.float32),
        pltpu.VMEM((CH, F), jnp.float32),
        pltpu.SemaphoreType.DMA,
        pltpu.SemaphoreType.DMA,
        pltpu.VMEM_SHARED((ACC_ROWS, F), jnp.float32),
    ],
    compiler_params=pltpu.CompilerParams(needs_layout_passes=False),
)
def _conv_kernel(csrc_h, cdst_h, z_h, out_h, si, di, r0, r1, sem0, sem1, acc):
    c = lax.axis_index("c")
    s = lax.axis_index("s")
    w = c * NS + s

    z16 = _z16()

    def _zero(i, _):
        r0[i // 8, pl.ds((i % 8) * L, L)] = z16
        return 0

    lax.fori_loop(0, CH * F // L, _zero, 0)

    for t in range(ZROWS // CH):
        pltpu.sync_copy(r0, acc.at[pl.ds(s * ZROWS + t * CH, CH)])
    plsc.subcore_barrier()

    def _group(g, _):
        base = w * CHUNKS + g * GSZ
        pltpu.sync_copy(csrc_h.at[pl.ds(base, GSZ)], si)
        pltpu.sync_copy(cdst_h.at[pl.ds(base, GSZ)], di)
        pltpu.async_copy(z_h.at[si.at[0]], r0, sem0)
        pltpu.async_copy(z_h.at[si.at[1]], r1, sem1)

        def _inner(t, _):
            k = 2 * t
            pltpu.make_async_copy(z_h.at[si.at[0]], r0, sem0).wait()
            pltpu.sync_copy(r0, acc.at[di.at[k]], add=True)

            @pl.when(k + 2 < GSZ)
            def _():
                pltpu.async_copy(z_h.at[si.at[k + 2]], r0, sem0)

            pltpu.make_async_copy(z_h.at[si.at[0]], r1, sem1).wait()
            pltpu.sync_copy(r1, acc.at[di.at[k + 1]], add=True)

            @pl.when(k + 3 < GSZ)
            def _():
                pltpu.async_copy(z_h.at[si.at[k + 3]], r1, sem1)

            return 0

        lax.fori_loop(0, GSZ // 2, _inner, 0)
        return 0

    lax.fori_loop(0, GROUPS, _group, 0)
    plsc.subcore_barrier()

    pltpu.sync_copy(acc.at[pl.ds(s * ZROWS, ZROWS)],
                    out_h.at[pl.ds(c * ACC_ROWS + s * ZROWS, ZROWS)])


def _norms(deg_ref):
    nsrc_a = lax.rsqrt(jnp.clip(deg_ref[:, 0:1], 1.0, None))
    ndst_a = lax.rsqrt(jnp.clip(deg_ref[:, 1:2], 1.0, None))
    nsrc_s = lax.rsqrt(jnp.clip(deg_ref[:, 2:3], 1.0, None))
    ndst_s = lax.rsqrt(jnp.clip(deg_ref[:, 3:4], 1.0, None))
    return nsrc_a, ndst_a, nsrc_s, ndst_s


def _dot(a, b):
    return jnp.dot(a, b, preferred_element_type=jnp.float32,
                   precision=lax.Precision.HIGHEST)


def _tc1_body(x_sex_ref, x_age_ref, deg_ref, wa_ref, ws_ref, out_ref):
    nsrc_a, _, nsrc_s, _ = _norms(deg_ref)
    out_ref[0:N] = _dot(x_sex_ref[...] * nsrc_a, wa_ref[...])
    out_ref[N:2 * N] = _dot(x_age_ref[...] * nsrc_s, ws_ref[...])


def _tc2_body(agg_ref, deg_ref, ba_ref, bs_ref, wa_ref, ws_ref, out_ref):
    nsrc_a, ndst_a, nsrc_s, ndst_s = _norms(deg_ref)
    h_age = jax.nn.relu(agg_ref[0:N] * ndst_a + ba_ref[...])
    h_sex = jax.nn.relu(agg_ref[OFF:OFF + N] * ndst_s + bs_ref[...])
    out_ref[0:N] = _dot(h_sex * nsrc_a, wa_ref[...])
    out_ref[N:2 * N] = _dot(h_age * nsrc_s, ws_ref[...])


def _tc3_body(agg_ref, deg_ref, ba_ref, bs_ref, oa_ref, os_ref):
    _, ndst_a, _, ndst_s = _norms(deg_ref)
    oa_ref[...] = agg_ref[0:N] * ndst_a + ba_ref[...]
    os_ref[...] = agg_ref[OFF:OFF + N] * ndst_s + bs_ref[...]


_zf = jax.ShapeDtypeStruct((2 * N, F), jnp.float32)
_of = jax.ShapeDtypeStruct((N, F), jnp.float32)
_af = jax.ShapeDtypeStruct((2 * ACC_ROWS, F), jnp.float32)

_tc1 = pl.pallas_call(_tc1_body, out_shape=_zf)
_tc2 = pl.pallas_call(_tc2_body, out_shape=_zf)
_tc3 = pl.pallas_call(_tc3_body, out_shape=(_of, _of))


def kernel(x_age, x_sex, edge_index_has_age, edge_index_has_sex,
           W1_age, b1_age, W1_sex, b1_sex,
           W2_age, b2_age, W2_sex, b2_sex):
    i32 = jnp.int32
    src_a, dst_a = edge_index_has_age[0], edge_index_has_age[1]
    src_s, dst_s = edge_index_has_sex[0], edge_index_has_sex[1]

    dsrc = jnp.concatenate([src_a, src_s]).reshape(NC * NS, DPT)
    ddst = jnp.concatenate([dst_a, dst_s]).reshape(NC * NS, DPT)

    npad = EPAD - E
    csrc = jnp.concatenate([
        src_a, jnp.zeros((npad,), i32),
        src_s + N, jnp.full((npad,), N, i32),
    ]).reshape(NC * NS * CHUNKS, CH)
    cdst = jnp.concatenate([
        dst_a, jnp.full((npad,), N, i32),
        dst_s, jnp.full((npad,), N, i32),
    ]).reshape(NC * NS * CHUNKS, CH)

    deg = _deg_kernel(dsrc, ddst)
    degt = deg.reshape(4, DROWS * DROWS)[:, :N].T

    ba1, bs1 = b1_age.reshape(1, F), b1_sex.reshape(1, F)
    ba2, bs2 = b2_age.reshape(1, F), b2_sex.reshape(1, F)

    z1 = _tc1(x_sex, x_age, degt, W1_age, W1_sex)
    agg1 = _conv_kernel(csrc, cdst, z1)
    z2 = _tc2(agg1, degt, ba1, bs1, W2_age, W2_sex)
    agg2 = _conv_kernel(csrc, cdst, z2)
    o_age, o_sex = _tc3(agg2, degt, ba2, bs2)
    return (o_age, o_sex)
